# baseline (speedup 1.0000x reference)
import functools

import jax
import jax.numpy as jnp
from jax.experimental import pallas as pl
from jax.experimental.pallas import tpu as pltpu

EPS = 1e-5
IMG_BLOCK = 8
TILE_M = 8192
VMEM_LIMIT = 100 * 1024 * 1024


def _im2col(xf, W):
    B, HW, C = xf.shape
    q = jax.lax.broadcasted_iota(jnp.int32, (HW, 1), 0)
    first_col = jax.lax.rem(q, W) == 0
    last_col = jax.lax.rem(q, W) == W - 1
    zero = jnp.zeros((), xf.dtype)
    cols = []
    for kh in range(3):
        for kw in range(3):
            s = (kh - 1) * W + (kw - 1)
            if s > 0:
                xs = jnp.concatenate(
                    [xf[:, s:, :], jnp.zeros((B, s, C), xf.dtype)], axis=1)
            elif s < 0:
                xs = jnp.concatenate(
                    [jnp.zeros((B, -s, C), xf.dtype), xf[:, :HW + s, :]],
                    axis=1)
            else:
                xs = xf
            if kw == 0:
                xs = jnp.where(first_col, zero, xs)
            elif kw == 2:
                xs = jnp.where(last_col, zero, xs)
            cols.append(xs)
    return jnp.concatenate(cols, axis=-1).reshape(B * HW, 9 * C)


def _finalize_bn(stats_ref, gamma_ref, beta_ref, M):
    stats = jnp.sum(stats_ref[...], axis=0)
    mean = stats[0:1] / M
    var = jnp.maximum(stats[1:2] / M - mean * mean, 0.0)
    scale = gamma_ref[...] * jax.lax.rsqrt(var + EPS)
    shift = beta_ref[...] - mean * scale
    return scale, shift


def _conv1_body(x_ref, w_ref, y_ref, xh_ref, stats_ref, *, W):
    B, C, H, _ = x_ref.shape
    HW = y_ref.shape[1]
    Cp = y_ref.shape[-1]

    eye = (jax.lax.broadcasted_iota(jnp.int32, (C, C), 0) ==
           jax.lax.broadcasted_iota(jnp.int32, (C, C), 1)).astype(jnp.bfloat16)
    xt = jax.lax.dot_general(
        x_ref[...].astype(jnp.bfloat16), eye,
        (((1,), (0,)), ((), ())),
        preferred_element_type=jnp.float32)
    xv = xt.astype(jnp.bfloat16).reshape(B, HW, C)
    xh_ref[...] = xv

    lhs = _im2col(xv, W)
    y = jnp.dot(lhs, w_ref[...], preferred_element_type=jnp.float32)

    y_ref[...] = y.reshape(B, HW, Cp).astype(y_ref.dtype)
    stats_ref[...] = jnp.concatenate(
        [jnp.sum(y, axis=0, keepdims=True),
         jnp.sum(y * y, axis=0, keepdims=True)], axis=0)


def _conv1(x_nchw, w_flat):
    N, C, H, W = x_nchw.shape
    HW = H * W
    Cp = w_flat.shape[-1]
    B = IMG_BLOCK if N % IMG_BLOCK == 0 else 1
    G = N // B
    flops = 2 * N * HW * (9 * C + C) * Cp
    bytes_accessed = (4 * x_nchw.size + 2 * w_flat.size
                      + 2 * N * HW * (Cp + C) + 4 * G * 2 * Cp)

    return pl.pallas_call(
        functools.partial(_conv1_body, W=W),
        out_shape=(jax.ShapeDtypeStruct((N, HW, Cp), jnp.bfloat16),
                   jax.ShapeDtypeStruct((N, HW, C), jnp.bfloat16),
                   jax.ShapeDtypeStruct((G, 2, Cp), jnp.float32)),
        grid=(G,),
        in_specs=[
            pl.BlockSpec((B, C, H, W), lambda g: (g, 0, 0, 0)),
            pl.BlockSpec((9 * C, Cp), lambda g: (0, 0)),
        ],
        out_specs=(
            pl.BlockSpec((B, HW, Cp), lambda g: (g, 0, 0)),
            pl.BlockSpec((B, HW, C), lambda g: (g, 0, 0)),
            pl.BlockSpec((None, 2, Cp), lambda g: (g, 0, 0)),
        ),
        compiler_params=pltpu.CompilerParams(
            dimension_semantics=("parallel",),
            vmem_limit_bytes=VMEM_LIMIT),
        cost_estimate=pl.CostEstimate(flops=flops, transcendentals=0,
                                      bytes_accessed=bytes_accessed),
    )(x_nchw, w_flat)


def _conv3x3_body(x_ref, w_ref, stats_in_ref, gamma_ref, beta_ref,
                  y_ref, stats_ref, *, pre_bn_relu, W, M):
    B, HW, C = x_ref.shape
    Cp = y_ref.shape[-1]

    if pre_bn_relu:
        scale, shift = _finalize_bn(stats_in_ref, gamma_ref, beta_ref, M)
        xv = x_ref[...].astype(jnp.float32)
        xv = jnp.maximum(xv * scale + shift, 0.0).astype(jnp.bfloat16)
    else:
        xv = x_ref[...].astype(jnp.bfloat16)

    lhs = _im2col(xv, W)
    y = jnp.dot(lhs, w_ref[...], preferred_element_type=jnp.float32)

    y_ref[...] = y.reshape(B, HW, Cp).astype(y_ref.dtype)
    stats_ref[...] = jnp.concatenate(
        [jnp.sum(y, axis=0, keepdims=True),
         jnp.sum(y * y, axis=0, keepdims=True)], axis=0)


def _conv3x3(x_nhwc, w_flat, W, prev_stats, gamma, beta, M):
    N, HW, C = x_nhwc.shape
    Cp = w_flat.shape[-1]
    pre = prev_stats is not None
    if not pre:
        prev_stats = jnp.zeros((1, 2, C), jnp.float32)
        gamma = jnp.ones((1, C), jnp.float32)
        beta = jnp.zeros((1, C), jnp.float32)
    Gp = prev_stats.shape[0]

    B = IMG_BLOCK if N % IMG_BLOCK == 0 else 1
    G = N // B
    flops = 2 * N * HW * (9 * C) * Cp
    bytes_accessed = (2 * x_nhwc.size + 2 * w_flat.size
                      + 2 * N * HW * Cp + 4 * G * 2 * Cp)

    return pl.pallas_call(
        functools.partial(_conv3x3_body, pre_bn_relu=pre, W=W, M=M),
        out_shape=(jax.ShapeDtypeStruct((N, HW, Cp), jnp.bfloat16),
                   jax.ShapeDtypeStruct((G, 2, Cp), jnp.float32)),
        grid=(G,),
        in_specs=[
            pl.BlockSpec((B, HW, C), lambda g: (g, 0, 0)),
            pl.BlockSpec((9 * C, Cp), lambda g: (0, 0)),
            pl.BlockSpec((Gp, 2, C), lambda g: (0, 0, 0)),
            pl.BlockSpec((1, C), lambda g: (0, 0)),
            pl.BlockSpec((1, C), lambda g: (0, 0)),
        ],
        out_specs=(
            pl.BlockSpec((B, HW, Cp), lambda g: (g, 0, 0)),
            pl.BlockSpec((None, 2, Cp), lambda g: (g, 0, 0)),
        ),
        compiler_params=pltpu.CompilerParams(
            dimension_semantics=("parallel",),
            vmem_limit_bytes=VMEM_LIMIT),
        cost_estimate=pl.CostEstimate(flops=flops, transcendentals=0,
                                      bytes_accessed=bytes_accessed),
    )(x_nhwc, w_flat, prev_stats, gamma, beta)


def _bn_add_relu_body(y_ref, res_ref, stats_ref, gamma_ref, beta_ref, o_ref,
                      *, M):
    C = o_ref.shape[-1]
    scale, shift = _finalize_bn(stats_ref, gamma_ref, beta_ref, M)
    y = y_ref[:, :C].astype(jnp.float32)
    res = res_ref[...].astype(jnp.float32)
    o_ref[...] = jnp.maximum(y * scale + shift + res, 0.0).astype(o_ref.dtype)


def _bn_add_relu(y2d, residual, stats, gamma, beta, M):
    Mrows, Cp = y2d.shape
    C = residual.shape[-1]
    G = stats.shape[0]
    tm = Mrows if Mrows <= TILE_M else TILE_M
    return pl.pallas_call(
        functools.partial(_bn_add_relu_body, M=M),
        out_shape=jax.ShapeDtypeStruct((Mrows, C), jnp.float32),
        grid=(pl.cdiv(Mrows, tm),),
        in_specs=[pl.BlockSpec((tm, Cp), lambda i: (i, 0)),
                  pl.BlockSpec((tm, C), lambda i: (i, 0)),
                  pl.BlockSpec((G, 2, Cp), lambda i: (0, 0, 0)),
                  pl.BlockSpec((1, C), lambda i: (0, 0)),
                  pl.BlockSpec((1, C), lambda i: (0, 0))],
        out_specs=pl.BlockSpec((tm, C), lambda i: (i, 0)),
        compiler_params=pltpu.CompilerParams(
            dimension_semantics=("parallel",),
            vmem_limit_bytes=VMEM_LIMIT),
    )(y2d, residual, stats, gamma, beta)


def _pack_w(w_oihw, cin_pad, cout_pad):
    cout, cin = w_oihw.shape[0], w_oihw.shape[1]
    w = jnp.transpose(w_oihw, (2, 3, 1, 0))
    w = jnp.pad(w, ((0, 0), (0, 0), (0, cin_pad - cin), (0, cout_pad - cout)))
    return w.reshape(9 * cin_pad, cout_pad).astype(jnp.bfloat16)


def kernel(x, w1, g1, b1, w2, g2, b2):
    N, Cin, H, W = x.shape
    Cout = w1.shape[0]
    Cp = ((Cout + 127) // 128) * 128
    M = N * H * W

    y1, xh, stats1 = _conv1(x, _pack_w(w1, Cin, Cp))
    y2, stats2 = _conv3x3(y1, _pack_w(w2, Cp, Cp), W,
                          stats1, g1.reshape(1, -1), b1.reshape(1, -1), M)

    out = _bn_add_relu(y2.reshape(M, Cp), xh.reshape(M, Cin),
                       stats2, g2.reshape(1, -1), b2.reshape(1, -1), M)
    return jnp.transpose(out.reshape(N, H, W, Cout), (0, 3, 1, 2))


# speedup vs baseline: 2.1364x; 2.1364x over previous
import functools

import jax
import jax.numpy as jnp
from jax.experimental import pallas as pl
from jax.experimental.pallas import tpu as pltpu

EPS = 1e-5
IMG_BLOCK = 8
TILE_M = 16384
VMEM_LIMIT = 100 * 1024 * 1024


def _im2col(xf, W):
    B, HW, C = xf.shape
    q = jax.lax.broadcasted_iota(jnp.int32, (HW, 1), 0)
    first_col = jax.lax.rem(q, W) == 0
    last_col = jax.lax.rem(q, W) == W - 1
    zero = jnp.zeros((), xf.dtype)
    cols = []
    for kh in range(3):
        for kw in range(3):
            s = (kh - 1) * W + (kw - 1)
            if s > 0:
                xs = jnp.concatenate(
                    [xf[:, s:, :], jnp.zeros((B, s, C), xf.dtype)], axis=1)
            elif s < 0:
                xs = jnp.concatenate(
                    [jnp.zeros((B, -s, C), xf.dtype), xf[:, :HW + s, :]],
                    axis=1)
            else:
                xs = xf
            if kw == 0:
                xs = jnp.where(first_col, zero, xs)
            elif kw == 2:
                xs = jnp.where(last_col, zero, xs)
            cols.append(xs)
    return jnp.concatenate(cols, axis=-1).reshape(B * HW, 9 * C)


def _finalize_bn(stats_ref, gamma_ref, beta_ref, M):
    stats = jnp.sum(stats_ref[...], axis=0)
    mean = stats[0:1] / M
    var = jnp.maximum(stats[1:2] / M - mean * mean, 0.0)
    scale = gamma_ref[...] * jax.lax.rsqrt(var + EPS)
    shift = beta_ref[...] - mean * scale
    return scale, shift


def _conv3x3_body(x_ref, w_ref, stats_in_ref, gamma_ref, beta_ref,
                  y_ref, stats_ref, *, pre_bn_relu, W, M):
    B, HW, C = x_ref.shape
    Cp = y_ref.shape[-1]

    if pre_bn_relu:
        scale, shift = _finalize_bn(stats_in_ref, gamma_ref, beta_ref, M)
        xv = x_ref[...].astype(jnp.float32)
        xv = jnp.maximum(xv * scale + shift, 0.0).astype(jnp.bfloat16)
    else:
        xv = x_ref[...].astype(jnp.bfloat16)

    lhs = _im2col(xv, W)
    y = jnp.dot(lhs, w_ref[...], preferred_element_type=jnp.float32)

    y_ref[...] = y.reshape(B, HW, Cp).astype(y_ref.dtype)
    stats_ref[...] = jnp.concatenate(
        [jnp.sum(y, axis=0, keepdims=True),
         jnp.sum(y * y, axis=0, keepdims=True)], axis=0)


def _conv3x3(x_nhwc, w_flat, W, prev_stats, gamma, beta, M):
    N, HW, C = x_nhwc.shape
    Cp = w_flat.shape[-1]
    pre = prev_stats is not None
    if not pre:
        prev_stats = jnp.zeros((1, 2, C), jnp.float32)
        gamma = jnp.ones((1, C), jnp.float32)
        beta = jnp.zeros((1, C), jnp.float32)
    Gp = prev_stats.shape[0]

    B = IMG_BLOCK if N % IMG_BLOCK == 0 else 1
    G = N // B
    flops = 2 * N * HW * (9 * C) * Cp
    bytes_accessed = (2 * x_nhwc.size + 2 * w_flat.size
                      + 2 * N * HW * Cp + 4 * G * 2 * Cp)

    return pl.pallas_call(
        functools.partial(_conv3x3_body, pre_bn_relu=pre, W=W, M=M),
        out_shape=(jax.ShapeDtypeStruct((N, HW, Cp), jnp.bfloat16),
                   jax.ShapeDtypeStruct((G, 2, Cp), jnp.float32)),
        grid=(G,),
        in_specs=[
            pl.BlockSpec((B, HW, C), lambda g: (g, 0, 0)),
            pl.BlockSpec((9 * C, Cp), lambda g: (0, 0)),
            pl.BlockSpec((Gp, 2, C), lambda g: (0, 0, 0)),
            pl.BlockSpec((1, C), lambda g: (0, 0)),
            pl.BlockSpec((1, C), lambda g: (0, 0)),
        ],
        out_specs=(
            pl.BlockSpec((B, HW, Cp), lambda g: (g, 0, 0)),
            pl.BlockSpec((None, 2, Cp), lambda g: (g, 0, 0)),
        ),
        compiler_params=pltpu.CompilerParams(
            dimension_semantics=("parallel",),
            vmem_limit_bytes=VMEM_LIMIT),
        cost_estimate=pl.CostEstimate(flops=flops, transcendentals=0,
                                      bytes_accessed=bytes_accessed),
    )(x_nhwc, w_flat, prev_stats, gamma, beta)


def _bn_add_relu_body(y_ref, res_ref, stats_ref, gamma_ref, beta_ref, o_ref,
                      *, M):
    C = o_ref.shape[-1]
    scale, shift = _finalize_bn(stats_ref, gamma_ref, beta_ref, M)
    y = y_ref[:, :C].astype(jnp.float32)
    res = res_ref[...].astype(jnp.float32)
    o_ref[...] = jnp.maximum(y * scale + shift + res, 0.0).astype(o_ref.dtype)


def _bn_add_relu(y2d, residual, stats, gamma, beta, M):
    Mrows, Cp = y2d.shape
    C = residual.shape[-1]
    G = stats.shape[0]
    tm = Mrows if Mrows <= TILE_M else TILE_M
    return pl.pallas_call(
        functools.partial(_bn_add_relu_body, M=M),
        out_shape=jax.ShapeDtypeStruct((Mrows, C), jnp.float32),
        grid=(pl.cdiv(Mrows, tm),),
        in_specs=[pl.BlockSpec((tm, Cp), lambda i: (i, 0)),
                  pl.BlockSpec((tm, C), lambda i: (i, 0)),
                  pl.BlockSpec((G, 2, Cp), lambda i: (0, 0, 0)),
                  pl.BlockSpec((1, C), lambda i: (0, 0)),
                  pl.BlockSpec((1, C), lambda i: (0, 0))],
        out_specs=pl.BlockSpec((tm, C), lambda i: (i, 0)),
        compiler_params=pltpu.CompilerParams(
            dimension_semantics=("parallel",),
            vmem_limit_bytes=VMEM_LIMIT),
    )(y2d, residual, stats, gamma, beta)


def _pack_w(w_oihw, cin_pad, cout_pad):
    cout, cin = w_oihw.shape[0], w_oihw.shape[1]
    w = jnp.transpose(w_oihw, (2, 3, 1, 0))
    w = jnp.pad(w, ((0, 0), (0, 0), (0, cin_pad - cin), (0, cout_pad - cout)))
    return w.reshape(9 * cin_pad, cout_pad).astype(jnp.bfloat16)


def kernel(x, w1, g1, b1, w2, g2, b2):
    N, Cin, H, W = x.shape
    Cout = w1.shape[0]
    Cp = ((Cout + 127) // 128) * 128
    M = N * H * W

    xh = jnp.transpose(x, (0, 2, 3, 1)).reshape(N, H * W, Cin)

    y1, stats1 = _conv3x3(xh, _pack_w(w1, Cin, Cp), W,
                          None, None, None, M)
    y2, stats2 = _conv3x3(y1, _pack_w(w2, Cp, Cp), W,
                          stats1, g1.reshape(1, -1), b1.reshape(1, -1), M)

    out = _bn_add_relu(y2.reshape(M, Cp), xh.reshape(M, Cin),
                       stats2, g2.reshape(1, -1), b2.reshape(1, -1), M)
    return jnp.transpose(out.reshape(N, H, W, Cout), (0, 3, 1, 2))
